# Initial kernel scaffold; baseline (speedup 1.0000x reference)
import jax
import jax.numpy as jnp
from jax.experimental import pallas as pl


def kernel(node_types, edge_index, edge_types, target_idx, label, node_table, edge_table, W_msg1, W_self1, W_agg1, W_msg2, W_self2, W_agg2, W_out, b_out):
    raise NotImplementedError("write your pallas kernel here")


# speedup vs baseline: 3.7149x; 3.7149x over previous
import functools

import jax
import jax.numpy as jnp
from jax import lax
from jax.experimental import pallas as pl
from jax.experimental.pallas import tpu as pltpu
from jax.experimental.pallas import tpu_sc as plsc

N = 10000
E = 160000
D = 256
DH = 128
NT = 32
ETY = 8
B = 64
RT = N * ETY

NC = 2
NS = 16
EP = 163840
EPW = EP // 32
EPT = EP // NS
K = 128
CH = EPT // K
NA = N + 16
ZR = 40
WB = 1000
PC = 1024


def _vec_mesh():
    return plsc.VectorSubcoreMesh(core_axis_name="c", subcore_axis_name="s")


@functools.partial(
    pl.kernel,
    out_type=jax.ShapeDtypeStruct((EP,), jnp.int32),
    mesh=_vec_mesh(),
    scratch_types=[
        pltpu.VMEM((PC,), jnp.int32),
        pltpu.VMEM((PC,), jnp.int32),
        pltpu.VMEM((PC,), jnp.int32),
    ],
)
def _prep(src_hbm, et_hbm, o_hbm, src_v, et_v, o_v):
    c = lax.axis_index("c")
    s = lax.axis_index("s")
    base = (s * NC + c) * EPW
    for i in range(EPW // PC):
        b = base + i * PC
        pltpu.sync_copy(src_hbm.at[pl.ds(b, PC)], src_v)
        pltpu.sync_copy(et_hbm.at[pl.ds(b, PC)], et_v)

        def body(j, _):
            sv = src_v[pl.ds(j * 16, 16)]
            ev = et_v[pl.ds(j * 16, 16)]
            o_v[pl.ds(j * 16, 16)] = sv * ETY + ev
            return 0

        lax.fori_loop(0, PC // 16, body, 0)
        pltpu.sync_copy(o_v, o_hbm.at[pl.ds(b, PC)])


@functools.partial(
    pl.kernel,
    out_type=jax.ShapeDtypeStruct((NC, N, DH), jnp.float32),
    mesh=_vec_mesh(),
    scratch_types=[
        pltpu.VMEM_SHARED((NA, DH), jnp.float32),
        pltpu.VMEM((CH, K), jnp.int32),
        pltpu.VMEM((CH, K), jnp.int32),
        pltpu.VMEM((K, DH), jnp.float32),
        pltpu.VMEM((ZR, DH), jnp.float32),
        pltpu.SemaphoreType.DMA,
    ],
)
def _edge_pass(table_hbm, idx_hbm, dst_hbm, out_hbm,
               acc, idx_v, dst_v, rows, zbuf, sem):
    c = lax.axis_index("c")
    s = lax.axis_index("s")

    def zrow(i, _):
        def zcol(j, _):
            zbuf[i, pl.ds(j * 16, 16)] = jnp.zeros((16,), jnp.float32)
            return 0
        lax.fori_loop(0, DH // 16, zcol, 0)
        return 0
    lax.fori_loop(0, ZR, zrow, 0)

    @pl.when(s < 10)
    def _():
        for z in range(WB // ZR):
            pltpu.sync_copy(zbuf, acc.at[pl.ds(s * WB + z * ZR, ZR)])

    @pl.when(s == 10)
    def _():
        pltpu.sync_copy(zbuf.at[pl.ds(0, NA - N)], acc.at[pl.ds(N, NA - N)])

    pltpu.sync_copy(idx_hbm.at[pl.ds(s * CH, CH)], idx_v)
    pltpu.sync_copy(dst_hbm.at[pl.ds(s * CH, CH)], dst_v)
    off = c * RT

    def orow(i, _):
        def ocol(j, _):
            idx_v[i, pl.ds(j * 16, 16)] = idx_v[i, pl.ds(j * 16, 16)] + off
            return 0
        lax.fori_loop(0, K // 16, ocol, 0)
        return 0
    lax.fori_loop(0, CH, orow, 0)
    plsc.subcore_barrier()

    def chunk(j, _):
        pltpu.async_copy(table_hbm.at[idx_v.at[j]], rows, sem).wait()
        pltpu.sync_copy(rows, acc.at[dst_v.at[j]], add=True)
        return 0
    lax.fori_loop(0, CH, chunk, 0)

    plsc.subcore_barrier()

    @pl.when(s < 10)
    def _():
        pltpu.sync_copy(acc.at[pl.ds(s * WB, WB)],
                        out_hbm.at[c, pl.ds(s * WB, WB)])


_dot = functools.partial(jnp.dot, preferred_element_type=jnp.float32,
                         precision=lax.Precision.HIGHEST)
NB = 1000
TB = NB * ETY


def _onehot_rows(nt_ref):
    nt = nt_ref[0, 0, :]
    return (nt[:, None] == lax.broadcasted_iota(jnp.int32, (NB, NT), 1)
            ).astype(jnp.float32)


def _build1_body(nt_ref, ntab_ref, wm1_ref, etab_ref, t1_ref):
    a1 = _dot(ntab_ref[...], wm1_ref[...])
    hw1 = _dot(_onehot_rows(nt_ref), a1)
    t1 = jax.nn.relu(hw1[:, None, :] + etab_ref[...][None, :, :])
    t1 = t1.reshape(TB, D)
    t1_ref[0] = t1[:, :DH]
    t1_ref[1] = t1[:, DH:]


def _build1(nt3, node_table, W_msg1, edge_table):
    return pl.pallas_call(
        _build1_body,
        grid=(N // NB,),
        in_specs=[
            pl.BlockSpec((1, 1, NB), lambda i: (i, 0, 0)),
            pl.BlockSpec((NT, D), lambda i: (0, 0)),
            pl.BlockSpec((D, D), lambda i: (0, 0)),
            pl.BlockSpec((ETY, D), lambda i: (0, 0)),
        ],
        out_specs=pl.BlockSpec((NC, TB, DH), lambda i: (0, i, 0)),
        out_shape=jax.ShapeDtypeStruct((NC, RT, DH), jnp.float32),
    )(nt3, node_table, W_msg1, edge_table)


def _dense1_body(nt_ref, agg_ref, ntab_ref, ws1_ref, wa1_ref, wm2_ref, etab_ref,
                 h1_ref, t2_ref):
    s1 = _dot(ntab_ref[...], ws1_ref[...])
    h0w = _dot(_onehot_rows(nt_ref), s1)
    agg = agg_ref[...]
    wa1 = wa1_ref[...]
    aggwa = (_dot(agg[0], wa1[:DH, :])
             + _dot(agg[1], wa1[DH:, :]))
    h1 = jax.nn.relu(h0w + aggwa)
    h1_ref[...] = h1
    hw2 = _dot(h1, wm2_ref[...])
    t2 = jax.nn.relu(hw2[:, None, :] + etab_ref[...][None, :, :])
    t2 = t2.reshape(TB, D)
    t2_ref[0] = t2[:, :DH]
    t2_ref[1] = t2[:, DH:]


def _dense1(nt3, agg1, node_table, W_self1, W_agg1, W_msg2, edge_table):
    return pl.pallas_call(
        _dense1_body,
        grid=(N // NB,),
        in_specs=[
            pl.BlockSpec((1, 1, NB), lambda i: (i, 0, 0)),
            pl.BlockSpec((NC, NB, DH), lambda i: (0, i, 0)),
            pl.BlockSpec((NT, D), lambda i: (0, 0)),
            pl.BlockSpec((D, D), lambda i: (0, 0)),
            pl.BlockSpec((D, D), lambda i: (0, 0)),
            pl.BlockSpec((D, D), lambda i: (0, 0)),
            pl.BlockSpec((ETY, D), lambda i: (0, 0)),
        ],
        out_specs=[
            pl.BlockSpec((NB, D), lambda i: (i, 0)),
            pl.BlockSpec((NC, TB, DH), lambda i: (0, i, 0)),
        ],
        out_shape=[
            jax.ShapeDtypeStruct((N, D), jnp.float32),
            jax.ShapeDtypeStruct((NC, RT, DH), jnp.float32),
        ],
    )(nt3, agg1, node_table, W_self1, W_agg1, W_msg2, edge_table)


def _dense2_body(h1_ref, agg_ref, ws2_ref, wa2_ref, wout_ref, b_ref,
                 tgt_ref, lab_ref, out_ref, zacc_ref):
    i = pl.program_id(0)
    h1 = h1_ref[...]
    agg = agg_ref[...]
    wa2 = wa2_ref[...]
    h2 = jax.nn.relu(
        _dot(h1, ws2_ref[...])
        + _dot(agg[0], wa2[:DH, :])
        + _dot(agg[1], wa2[DH:, :]))
    s = _dot(h2, wout_ref[...])
    ga = lax.broadcasted_iota(jnp.int32, (B, 8), 0)
    gb = lax.broadcasted_iota(jnp.int32, (B, 8), 1)
    rsel = (ga // 8 == gb).astype(jnp.float32)
    csel = (ga % 8 == gb).astype(jnp.float32)
    tflat = jnp.sum(_dot(rsel, tgt_ref[...].astype(jnp.float32)) * csel,
                    axis=1, keepdims=True)
    oh = (tflat.astype(jnp.int32) - i * NB
          == lax.broadcasted_iota(jnp.int32, (B, NB), 1))
    zpart = _dot(oh.astype(jnp.float32), s)

    @pl.when(i == 0)
    def _():
        zacc_ref[...] = jnp.zeros((B, 1), jnp.float32)

    zacc_ref[...] += zpart

    @pl.when(i == N // NB - 1)
    def _():
        lab = jnp.sum(_dot(rsel, lab_ref[...]) * csel,
                      axis=1, keepdims=True)
        z = zacc_ref[...] + b_ref[...]
        p = jax.nn.sigmoid(z)
        loss = (-lab * jnp.log(p + 1e-18)
                - (1.0 - lab) * jnp.log(1.0 - p + 1e-18))
        out_ref[...] = jnp.mean(loss).reshape(1, 1)


def _dense2(h1, agg2, W_self2, W_agg2, W_out, b2, tgt, lab):
    return pl.pallas_call(
        _dense2_body,
        grid=(N // NB,),
        in_specs=[
            pl.BlockSpec((NB, D), lambda i: (i, 0)),
            pl.BlockSpec((NC, NB, DH), lambda i: (0, i, 0)),
            pl.BlockSpec((D, D), lambda i: (0, 0)),
            pl.BlockSpec((D, D), lambda i: (0, 0)),
            pl.BlockSpec((D, 1), lambda i: (0, 0)),
            pl.BlockSpec((1, 1), lambda i: (0, 0)),
            pl.BlockSpec((8, 8), lambda i: (0, 0)),
            pl.BlockSpec((8, 8), lambda i: (0, 0)),
        ],
        out_specs=pl.BlockSpec((1, 1), lambda i: (0, 0)),
        out_shape=jax.ShapeDtypeStruct((1, 1), jnp.float32),
        scratch_shapes=[pltpu.VMEM((B, 1), jnp.float32)],
    )(h1, agg2, W_self2, W_agg2, W_out, b2, tgt, lab)


def kernel(node_types, edge_index, edge_types, target_idx, label,
           node_table, edge_table,
           W_msg1, W_self1, W_agg1, W_msg2, W_self2, W_agg2,
           W_out, b_out):
    nt = node_types.astype(jnp.int32)
    src = edge_index[0].astype(jnp.int32)
    dst = edge_index[1].astype(jnp.int32)
    et = edge_types.astype(jnp.int32)

    pad = EP - E
    src_p = jnp.concatenate([src, jnp.zeros((pad,), jnp.int32)])
    et_p = jnp.concatenate([et, jnp.zeros((pad,), jnp.int32)])
    dst_p = jnp.concatenate([dst, jnp.full((pad,), N, jnp.int32)])

    nt3 = nt.reshape(N // NB, 1, NB)
    idx = _prep(src_p, et_p)
    t1 = _build1(nt3, node_table, W_msg1, edge_table)

    idx2 = idx.reshape(NS * CH, K)
    dst2 = dst_p.reshape(NS * CH, K)
    agg1 = _edge_pass(t1.reshape(NC * RT, DH), idx2, dst2)

    h1, t2 = _dense1(nt3, agg1, node_table, W_self1, W_agg1, W_msg2, edge_table)

    agg2 = _edge_pass(t2.reshape(NC * RT, DH), idx2, dst2)

    loss = _dense2(h1, agg2, W_self2, W_agg2, W_out,
                   b_out.reshape(1, 1).astype(jnp.float32),
                   target_idx.astype(jnp.int32).reshape(8, 8),
                   label.astype(jnp.float32).reshape(8, 8))
    return loss[0, 0]
